# baseline (device time: 22966 ns/iter reference)
import jax
import jax.numpy as jnp
from jax import lax
from jax.experimental import pallas as pl
from jax.experimental.pallas import tpu as pltpu

CM = 64
CQ = 8
CH = CQ // 2


def kernel(x):
    m_per, n = x.shape
    qr = m_per // 4

    def body(
        x_hbm, out_hbm,
        in_v, own_bf, fland, yland, zland,
        lin, linq, lown, lout, lyz,
        xs, xr, ys, yr, zs, zr,
    ):
        my_x = lax.axis_index("x")
        my_y = lax.axis_index("y")
        my_z = lax.axis_index("z")
        peer_x = (1 - my_x, my_y, my_z)
        peer_y = (my_x, 1 - my_y, my_z)
        peer_z = (my_x, my_y, 1 - my_z)

        my_idx = 2 * my_y + my_z
        d_y = 2 * my_y + (1 - my_z)
        d_z = 2 * (1 - my_y) + my_z
        diag = 2 * (1 - my_y) + (1 - my_z)

        own_base = my_x * m_per
        far_base = (1 - my_x) * m_per

        lin_cp = []
        for c in range(CQ):
            rows = pl.ds(my_idx * qr + c * CM, CM)
            cp = pltpu.make_async_copy(x_hbm.at[rows, :], in_v.at[rows, :],
                                       lin.at[c])
            cp.start()
            lin_cp.append(cp)
        linq_cp = []
        for q in range(4):
            rows = pl.ds(q * qr, qr)
            cp = pltpu.make_async_copy(x_hbm.at[rows, :], in_v.at[rows, :],
                                       linq.at[q])
            cp.start()
            linq_cp.append(cp)

        barrier_sem = pltpu.get_barrier_semaphore()
        for p in (peer_x, peer_y, peer_z):
            pl.semaphore_signal(
                barrier_sem, inc=1, device_id=p,
                device_id_type=pl.DeviceIdType.MESH,
            )
        pl.semaphore_wait(barrier_sem, 3)

        x_out = []
        for c in range(CQ):
            lin_cp[c].wait()
            rows = pl.ds(my_idx * qr + c * CM, CM)
            own_bf[rows, :] = in_v[rows, :].astype(jnp.bfloat16)
            r = pltpu.make_async_remote_copy(
                src_ref=own_bf.at[rows, :],
                dst_ref=fland.at[pl.ds(c * CM, CM), :],
                send_sem=xs.at[c], recv_sem=xr.at[c],
                device_id=peer_x, device_id_type=pl.DeviceIdType.MESH,
            )
            r.start()
            x_out.append(r)

        for cp in linq_cp:
            cp.wait()
        own_bf[...] = in_v[...].astype(jnp.bfloat16)
        cp_own = pltpu.make_async_copy(
            own_bf, out_hbm.at[pl.ds(own_base, m_per), :], lown
        )
        cp_own.start()

        y_out, z_out, lout_cp = [], [], []
        for c in range(CQ):
            land = fland.at[pl.ds(c * CM, CM), :]
            pltpu.make_async_remote_copy(
                src_ref=land, dst_ref=land,
                send_sem=xs.at[c], recv_sem=xr.at[c],
                device_id=peer_x, device_id_type=pl.DeviceIdType.MESH,
            ).wait_recv()
            ry = pltpu.make_async_remote_copy(
                src_ref=land, dst_ref=yland.at[pl.ds(c * CM, CM), :],
                send_sem=ys.at[c], recv_sem=yr.at[c],
                device_id=peer_y, device_id_type=pl.DeviceIdType.MESH,
            )
            ry.start()
            y_out.append(ry)
            rz = pltpu.make_async_remote_copy(
                src_ref=land, dst_ref=zland.at[pl.ds(c * CM, CM), :],
                send_sem=zs.at[c], recv_sem=zr.at[c],
                device_id=peer_z, device_id_type=pl.DeviceIdType.MESH,
            )
            rz.start()
            z_out.append(rz)
            cp = pltpu.make_async_copy(
                land, out_hbm.at[pl.ds(far_base + my_idx * qr + c * CM, CM), :],
                lout,
            )
            cp.start()
            lout_cp.append(cp)

        for k in range(CH):
            zin = zland.at[pl.ds(k * CM, CM), :]
            pltpu.make_async_remote_copy(
                src_ref=zin, dst_ref=zin,
                send_sem=zs.at[k], recv_sem=zr.at[k],
                device_id=peer_z, device_id_type=pl.DeviceIdType.MESH,
            ).wait_recv()
            ry = pltpu.make_async_remote_copy(
                src_ref=zin,
                dst_ref=out_hbm.at[pl.ds(far_base + d_y * qr + k * CM, CM), :],
                send_sem=ys.at[CQ + k], recv_sem=yr.at[CQ + k],
                device_id=peer_y, device_id_type=pl.DeviceIdType.MESH,
            )
            ry.start()
            y_out.append(ry)
        for k in range(CH):
            yin = yland.at[pl.ds((CH + k) * CM, CM), :]
            pltpu.make_async_remote_copy(
                src_ref=yin, dst_ref=yin,
                send_sem=ys.at[CH + k], recv_sem=yr.at[CH + k],
                device_id=peer_y, device_id_type=pl.DeviceIdType.MESH,
            ).wait_recv()
            rz = pltpu.make_async_remote_copy(
                src_ref=yin,
                dst_ref=out_hbm.at[
                    pl.ds(far_base + d_z * qr + (CH + k) * CM, CM), :
                ],
                send_sem=zs.at[CQ + k], recv_sem=zr.at[CQ + k],
                device_id=peer_z, device_id_type=pl.DeviceIdType.MESH,
            )
            rz.start()
            z_out.append(rz)

        for k in range(CH):
            pltpu.make_async_remote_copy(
                src_ref=yland.at[pl.ds(k * CM, CM), :],
                dst_ref=yland.at[pl.ds(k * CM, CM), :],
                send_sem=ys.at[k], recv_sem=yr.at[k],
                device_id=peer_y, device_id_type=pl.DeviceIdType.MESH,
            ).wait_recv()
        cp_y = pltpu.make_async_copy(
            yland, out_hbm.at[pl.ds(far_base + d_z * qr, qr), :], lyz
        )
        cp_y.start()
        for k in range(CH):
            pltpu.make_async_remote_copy(
                src_ref=zland.at[pl.ds((CH + k) * CM, CM), :],
                dst_ref=zland.at[pl.ds((CH + k) * CM, CM), :],
                send_sem=zs.at[CH + k], recv_sem=zr.at[CH + k],
                device_id=peer_z, device_id_type=pl.DeviceIdType.MESH,
            ).wait_recv()
        cp_z = pltpu.make_async_copy(
            zland, out_hbm.at[pl.ds(far_base + d_y * qr, qr), :], lyz
        )
        cp_z.start()

        for k in range(CH):
            pltpu.make_async_remote_copy(
                src_ref=yland.at[pl.ds(k * CM, CM), :],
                dst_ref=out_hbm.at[pl.ds(far_base + diag * qr + k * CM, CM), :],
                send_sem=ys.at[CQ + k], recv_sem=yr.at[CQ + k],
                device_id=peer_y, device_id_type=pl.DeviceIdType.MESH,
            ).wait_recv()
        for k in range(CH):
            pltpu.make_async_remote_copy(
                src_ref=zland.at[pl.ds(k * CM, CM), :],
                dst_ref=out_hbm.at[
                    pl.ds(far_base + diag * qr + qr // 2 + k * CM, CM), :
                ],
                send_sem=zs.at[CQ + k], recv_sem=zr.at[CQ + k],
                device_id=peer_z, device_id_type=pl.DeviceIdType.MESH,
            ).wait_recv()

        for r in x_out + y_out + z_out:
            r.wait_send()
        cp_own.wait()
        for cp in lout_cp:
            cp.wait()
        cp_y.wait()
        cp_z.wait()

    return pl.pallas_call(
        body,
        out_shape=jax.ShapeDtypeStruct((2 * m_per, n), jnp.bfloat16),
        in_specs=[pl.BlockSpec(memory_space=pltpu.MemorySpace.HBM)],
        out_specs=pl.BlockSpec(memory_space=pltpu.MemorySpace.HBM),
        scratch_shapes=[
            pltpu.VMEM((m_per, n), x.dtype),
            pltpu.VMEM((m_per, n), jnp.bfloat16),
            pltpu.VMEM((qr, n), jnp.bfloat16),
            pltpu.VMEM((qr, n), jnp.bfloat16),
            pltpu.VMEM((qr, n), jnp.bfloat16),
            pltpu.SemaphoreType.DMA((CQ,)),
            pltpu.SemaphoreType.DMA((4,)),
            pltpu.SemaphoreType.DMA,
            pltpu.SemaphoreType.DMA,
            pltpu.SemaphoreType.DMA,
            pltpu.SemaphoreType.DMA((CQ,)),
            pltpu.SemaphoreType.DMA((CQ,)),
            pltpu.SemaphoreType.DMA((CQ + CH,)),
            pltpu.SemaphoreType.DMA((CQ + CH,)),
            pltpu.SemaphoreType.DMA((CQ + CH,)),
            pltpu.SemaphoreType.DMA((CQ + CH,)),
        ],
        compiler_params=pltpu.CompilerParams(collective_id=0),
    )(x)
